# baseline (speedup 1.0000x reference)
import functools

import jax
import jax.numpy as jnp
from jax import lax
from jax.experimental import pallas as pl
from jax.experimental.pallas import tpu as pltpu
from jax.experimental.pallas import tpu_sc as plsc

N2 = 10240
D = 128
NC = 2
NS = 16
EROWS = 2500
ROWS_PER_CORE = EROWS // NC
NPT = N2 // NS
BLK = 512
GRID = N2 // BLK


def _dot_t(a, w):
    return lax.dot_general(a, w, (((1,), (1,)), ((), ())),
                           preferred_element_type=jnp.float32)


def _k1_body(x_ref, wl_ref, wr_ref, b_ref, y_ref, r_ref):
    xb = x_ref[...]
    y_ref[...] = _dot_t(xb, wl_ref[...])
    r_ref[...] = _dot_t(xb, wr_ref[...]) + b_ref[...][None, :]


def _k2_body(agg_ref, cnt_ref, r1_ref, wl_ref, wr_ref, b_ref,
             y2_ref, r2_ref, inv_ref):
    cnt = cnt_ref[0] + cnt_ref[1]
    inv = 1.0 / jnp.maximum(cnt, 1.0)
    mean = (agg_ref[0] + agg_ref[1]) * inv[:, None]
    h = jnp.maximum(mean + r1_ref[...], 0.0)
    y2_ref[...] = _dot_t(h, wl_ref[...])
    r2_ref[...] = _dot_t(h, wr_ref[...]) + b_ref[...][None, :]
    inv_ref[...] = inv


def _k3_body(agg_ref, inv_ref, r2_ref, out_ref):
    out_ref[...] = (agg_ref[0] + agg_ref[1]) * inv_ref[...][:, None] \
        + r2_ref[...]


_w_spec = pl.BlockSpec((D, D), lambda i: (0, 0))
_b_spec = pl.BlockSpec((D,), lambda i: (0,))
_row_spec = pl.BlockSpec((BLK, D), lambda i: (i, 0))
_p_spec = pl.BlockSpec((2, BLK, D), lambda i: (0, i, 0))
_c_spec = pl.BlockSpec((2, BLK), lambda i: (0, i))
_v_spec = pl.BlockSpec((BLK,), lambda i: (i,))

_k1 = pl.pallas_call(
    _k1_body,
    grid=(GRID,),
    in_specs=[_row_spec, _w_spec, _w_spec, _b_spec],
    out_specs=[_row_spec, _row_spec],
    out_shape=[jax.ShapeDtypeStruct((N2, D), jnp.float32)] * 2,
)

_k2 = pl.pallas_call(
    _k2_body,
    grid=(GRID,),
    in_specs=[_p_spec, _c_spec, _row_spec, _w_spec, _w_spec, _b_spec],
    out_specs=[_row_spec, _row_spec, _v_spec],
    out_shape=[jax.ShapeDtypeStruct((N2, D), jnp.float32),
               jax.ShapeDtypeStruct((N2, D), jnp.float32),
               jax.ShapeDtypeStruct((N2,), jnp.float32)],
)

_k3 = pl.pallas_call(
    _k3_body,
    grid=(GRID,),
    in_specs=[_p_spec, _v_spec, _row_spec],
    out_specs=_row_spec,
    out_shape=jax.ShapeDtypeStruct((N2, D), jnp.float32),
)


def _agg_body(with_cnt, y_hbm, src_hbm, dst_hbm, z_hbm, *refs):
    if with_cnt:
        agg_out, cnt_out = refs[0], refs[1]
        rest = refs[2:]
    else:
        agg_out = refs[0]
        rest = refs[1:]
    rows0, sidx, didx, ones, cbuf, agg_sh, cnt_sh = rest

    c = lax.axis_index("c")
    t = lax.axis_index("s")

    if with_cnt:
        for k in range(8):
            ones[pl.ds(k * 16, 16)] = jnp.ones((16,), jnp.float32)

    for k in range(NPT // 128):
        pltpu.sync_copy(z_hbm, agg_sh.at[pl.ds(t * NPT + k * 128, 128)])
        if with_cnt:
            pltpu.sync_copy(z_hbm.at[0],
                            cnt_sh.at[pl.ds(t * NPT + k * 128, 128)])
    plsc.subcore_barrier()

    base = c * ROWS_PER_CORE
    nrows = jnp.where(t < ROWS_PER_CORE % NS, ROWS_PER_CORE // NS + 1,
                      ROWS_PER_CORE // NS)

    def row_step(g, cc):
        r = base + t + g * NS
        pltpu.sync_copy(src_hbm.at[r], sidx.at[0])
        pltpu.sync_copy(dst_hbm.at[r], didx.at[0])
        pltpu.sync_copy(y_hbm.at[sidx.at[0]], rows0)
        pltpu.sync_copy(rows0, agg_sh.at[didx.at[0]], add=True)
        if with_cnt:
            pltpu.sync_copy(ones, cnt_sh.at[didx.at[0]], add=True)
        return cc

    lax.fori_loop(0, nrows, row_step, 0)
    plsc.subcore_barrier()

    for k in range(NPT // 128):
        sl = pl.ds(t * NPT + k * 128, 128)
        pltpu.sync_copy(agg_sh.at[sl], agg_out.at[c, sl])
        if with_cnt:
            pltpu.sync_copy(cnt_sh.at[sl], cnt_out.at[c, sl])


def _make_agg(with_cnt):
    out_type = [jax.ShapeDtypeStruct((NC, N2, D), jnp.float32)]
    if with_cnt:
        out_type.append(jax.ShapeDtypeStruct((NC, N2), jnp.float32))
    return pl.kernel(
        functools.partial(_agg_body, with_cnt),
        out_type=out_type,
        mesh=plsc.VectorSubcoreMesh(core_axis_name="c", subcore_axis_name="s",
                                    num_cores=NC, num_subcores=NS),
        scratch_types=[
            pltpu.VMEM((128, D), jnp.float32),
            pltpu.VMEM((1, 128), jnp.int32),
            pltpu.VMEM((1, 128), jnp.int32),
            pltpu.VMEM((128,), jnp.float32),
            pltpu.VMEM((128,), jnp.float32),
            pltpu.VMEM_SHARED((N2, D), jnp.float32),
            pltpu.VMEM_SHARED((N2,), jnp.float32),
        ],
    )


_agg_cnt = _make_agg(True)
_agg = _make_agg(False)


@jax.jit
def kernel(x, edge_index, edge_attr, W1l, b1l, W1r, W2l, b2l, W2r):
    del edge_attr
    n = x.shape[0]
    xp = jnp.pad(x, ((0, N2 - n), (0, 0)))
    src = edge_index[0].astype(jnp.int32).reshape(EROWS, 128)
    dst = edge_index[1].astype(jnp.int32).reshape(EROWS, 128)
    zeros = jnp.zeros((128, D), jnp.float32)

    y1, r1 = _k1(xp, W1l, W1r, b1l)
    p1, cnt = _agg_cnt(y1, src, dst, zeros)
    y2, r2, inv = _k2(p1, cnt, r1, W2l, W2r, b2l)
    p2, = _agg(y2, src, dst, zeros)
    out = _k3(p2, inv, r2)
    return out[:n]


# speedup vs baseline: 1.3397x; 1.3397x over previous
import functools

import jax
import jax.numpy as jnp
from jax import lax
from jax.experimental import pallas as pl
from jax.experimental.pallas import tpu as pltpu
from jax.experimental.pallas import tpu_sc as plsc

N2 = 10240
D = 128
NC = 2
NS = 16
EROWS = 2500
ROWS_PER_CORE = EROWS // NC
NPT = N2 // NS
BLK = 512
GRID = N2 // BLK


def _dot_t(a, w):
    return lax.dot_general(a, w, (((1,), (1,)), ((), ())),
                           preferred_element_type=jnp.float32)


def _k1_body(x_ref, wl_ref, wr_ref, b_ref, y_ref, r_ref):
    xb = x_ref[...]
    y_ref[...] = _dot_t(xb, wl_ref[...])
    r_ref[...] = _dot_t(xb, wr_ref[...]) + b_ref[...][None, :]


def _k2_body(agg_ref, cnt_ref, r1_ref, wl_ref, wr_ref, b_ref,
             y2_ref, r2_ref, inv_ref):
    cnt = cnt_ref[0] + cnt_ref[1]
    inv = 1.0 / jnp.maximum(cnt, 1.0)
    mean = (agg_ref[0] + agg_ref[1]) * inv[:, None]
    h = jnp.maximum(mean + r1_ref[...], 0.0)
    y2_ref[...] = _dot_t(h, wl_ref[...])
    r2_ref[...] = _dot_t(h, wr_ref[...]) + b_ref[...][None, :]
    inv_ref[...] = inv


def _k3_body(agg_ref, inv_ref, r2_ref, out_ref):
    out_ref[...] = (agg_ref[0] + agg_ref[1]) * inv_ref[...][:, None] \
        + r2_ref[...]


_w_spec = pl.BlockSpec((D, D), lambda i: (0, 0))
_b_spec = pl.BlockSpec((D,), lambda i: (0,))
_row_spec = pl.BlockSpec((BLK, D), lambda i: (i, 0))
_p_spec = pl.BlockSpec((2, BLK, D), lambda i: (0, i, 0))
_c_spec = pl.BlockSpec((2, BLK), lambda i: (0, i))
_v_spec = pl.BlockSpec((BLK,), lambda i: (i,))

_k1 = pl.pallas_call(
    _k1_body,
    grid=(GRID,),
    in_specs=[_row_spec, _w_spec, _w_spec, _b_spec],
    out_specs=[_row_spec, _row_spec],
    out_shape=[jax.ShapeDtypeStruct((N2, D), jnp.float32)] * 2,
)

_k2 = pl.pallas_call(
    _k2_body,
    grid=(GRID,),
    in_specs=[_p_spec, _c_spec, _row_spec, _w_spec, _w_spec, _b_spec],
    out_specs=[_row_spec, _row_spec, _v_spec],
    out_shape=[jax.ShapeDtypeStruct((N2, D), jnp.float32),
               jax.ShapeDtypeStruct((N2, D), jnp.float32),
               jax.ShapeDtypeStruct((N2,), jnp.float32)],
)

_k3 = pl.pallas_call(
    _k3_body,
    grid=(GRID,),
    in_specs=[_p_spec, _v_spec, _row_spec],
    out_specs=_row_spec,
    out_shape=jax.ShapeDtypeStruct((N2, D), jnp.float32),
)


def _agg_body(with_cnt, y_hbm, src_hbm, dst_hbm, z_hbm, *refs):
    if with_cnt:
        agg_out, cnt_out = refs[0], refs[1]
        rest = refs[2:]
    else:
        agg_out = refs[0]
        rest = refs[1:]
    rows0, sidx, didx, ones, cbuf, isem, agg_sh, cnt_sh = rest

    c = lax.axis_index("c")
    t = lax.axis_index("s")

    if with_cnt:
        for k in range(8):
            ones[pl.ds(k * 16, 16)] = jnp.ones((16,), jnp.float32)

    pltpu.sync_copy(z_hbm, rows0)
    for k in range(NPT // 128):
        pltpu.sync_copy(rows0, agg_sh.at[pl.ds(t * NPT + k * 128, 128)])
        if with_cnt:
            pltpu.sync_copy(rows0.at[0],
                            cnt_sh.at[pl.ds(t * NPT + k * 128, 128)])
    plsc.subcore_barrier()

    base = c * ROWS_PER_CORE
    rem = ROWS_PER_CORE % NS
    npairs = (ROWS_PER_CORE // NS) // 2

    def _row(g):
        return base + t + g * NS

    def prefetch(g, b):
        r = _row(g)

        @pl.when(r < base + ROWS_PER_CORE)
        def _():
            pltpu.async_copy(src_hbm.at[r], sidx.at[b], isem)
            pltpu.async_copy(dst_hbm.at[r], didx.at[b], isem)

    def drain(g, b):
        r = _row(g)

        @pl.when(r < base + ROWS_PER_CORE)
        def _():
            pltpu.make_async_copy(src_hbm.at[r], sidx.at[b], isem).wait()
            pltpu.make_async_copy(dst_hbm.at[r], didx.at[b], isem).wait()

    def work(b):
        pltpu.sync_copy(y_hbm.at[sidx.at[b]], rows0)
        pltpu.sync_copy(rows0, agg_sh.at[didx.at[b]], add=True)
        if with_cnt:
            pltpu.sync_copy(ones, cnt_sh.at[didx.at[b]], add=True)

    pltpu.sync_copy(src_hbm.at[_row(0)], sidx.at[0])
    pltpu.sync_copy(dst_hbm.at[_row(0)], didx.at[0])

    def pair_step(p, cc):
        prefetch(2 * p + 1, 1)
        work(0)
        drain(2 * p + 1, 1)
        prefetch(2 * p + 2, 0)
        work(1)
        drain(2 * p + 2, 0)
        return cc

    lax.fori_loop(0, npairs, pair_step, 0)

    @pl.when(t < rem)
    def _():
        work(0)
    plsc.subcore_barrier()

    for k in range(NPT // 128):
        sl = pl.ds(t * NPT + k * 128, 128)
        pltpu.sync_copy(agg_sh.at[sl], rows0)
        pltpu.sync_copy(rows0, agg_out.at[c, sl])
        if with_cnt:
            pltpu.sync_copy(cnt_sh.at[sl], cbuf)
            pltpu.sync_copy(cbuf, cnt_out.at[c, sl])


def _make_agg(with_cnt):
    out_type = [jax.ShapeDtypeStruct((NC, N2, D), jnp.float32)]
    if with_cnt:
        out_type.append(jax.ShapeDtypeStruct((NC, N2), jnp.float32))
    return pl.kernel(
        functools.partial(_agg_body, with_cnt),
        out_type=out_type,
        mesh=plsc.VectorSubcoreMesh(core_axis_name="c", subcore_axis_name="s",
                                    num_cores=NC, num_subcores=NS),
        scratch_types=[
            pltpu.VMEM((128, D), jnp.float32),
            pltpu.VMEM((2, 128), jnp.int32),
            pltpu.VMEM((2, 128), jnp.int32),
            pltpu.VMEM((128,), jnp.float32),
            pltpu.VMEM((128,), jnp.float32),
            pltpu.SemaphoreType.DMA,
            pltpu.VMEM_SHARED((N2, D), jnp.float32),
            pltpu.VMEM_SHARED((N2,), jnp.float32),
        ],
    )


_agg_cnt = _make_agg(True)
_agg = _make_agg(False)


@jax.jit
def kernel(x, edge_index, edge_attr, W1l, b1l, W1r, W2l, b2l, W2r):
    del edge_attr
    n = x.shape[0]
    xp = jnp.pad(x, ((0, N2 - n), (0, 0)))
    src = edge_index[0].astype(jnp.int32).reshape(EROWS, 128)
    dst = edge_index[1].astype(jnp.int32).reshape(EROWS, 128)
    zeros = jnp.zeros((128, D), jnp.float32)

    y1, r1 = _k1(xp, W1l, W1r, b1l)
    p1, cnt = _agg_cnt(y1, src, dst, zeros)
    y2, r2, inv = _k2(p1, cnt, r1, W2l, W2r, b2l)
    p2, = _agg(y2, src, dst, zeros)
    out = _k3(p2, inv, r2)
    return out[:n]
